# baseline (speedup 1.0000x reference)
import functools

import numpy as np
import jax
import jax.numpy as jnp
from jax.experimental import pallas as pl
from jax.experimental.pallas import tpu as pltpu

_NBINS = 100
_CUTOFF = 0.35
_NA = 500
_NAP = 512
_CB = _CUTOFF + 0.5
_INVW = (_NBINS - 1) / _CUTOFF


def _tc_body(rows_ref, cols_ref, out_ref):
    a = rows_ref[0]
    c = cols_ref[0]
    xr, yr, zr = a[0:1, :], a[1:2, :], a[2:3, :]
    xc, yc, zc = c[:, 0:1], c[:, 1:2], c[:, 2:3]

    def mim(d):
        return (d - jnp.where(d >= 0.5, 1.0, 0.0)
                  + jnp.where(d < -0.5, 1.0, 0.0))

    dx = mim(xc - xr)
    dy = mim(yc - yr)
    dz = mim(zc - zr)
    dsq = dx * dx + dy * dy + dz * dz
    mask = (dsq < _CB * _CB) & (dsq != 0.0)
    mf = mask.astype(jnp.float32)
    d = jnp.sqrt(jnp.where(mask, dsq, 1.0))
    t = d * _INVW

    def body(k, acc):
        kf = k.astype(jnp.float32)
        e = jnp.exp(-0.5 * (t - kf) ** 2) * mf
        s = jnp.sum(e)
        onehot = (jax.lax.broadcasted_iota(jnp.int32, (1, 128), 1)
                  == k).astype(jnp.float32)
        return acc + s * onehot

    acc = jax.lax.fori_loop(0, _NBINS, body,
                            jnp.zeros((1, 128), jnp.float32))
    out_ref[0] = acc


def _histogram(xyz):
    b = xyz.shape[0]
    pad = jnp.full((b, _NAP - _NA, 3), jnp.nan, jnp.float32)
    xyzp = jnp.concatenate([xyz, pad], axis=1)
    rows = jnp.transpose(xyzp, (0, 2, 1))
    cols = jnp.pad(xyzp, ((0, 0), (0, 0), (0, 5)))

    partial = pl.pallas_call(
        _tc_body,
        grid=(b,),
        in_specs=[
            pl.BlockSpec((1, 3, _NAP), lambda i: (i, 0, 0)),
            pl.BlockSpec((1, _NAP, 8), lambda i: (i, 0, 0)),
        ],
        out_specs=pl.BlockSpec((1, 1, 128), lambda i: (i, 0, 0)),
        out_shape=jax.ShapeDtypeStruct((b, 1, 128), jnp.float32),
    )(rows, cols)
    return partial.sum(axis=0)[0, :_NBINS]


def kernel(xyz):
    count = _histogram(xyz)
    bins = jnp.linspace(0.0, _CUTOFF, _NBINS + 1)
    vol_bins = 4.0 * np.pi / 3.0 * (bins[1:] ** 3 - bins[:-1] ** 3)
    norm = count.sum()
    count = count / norm
    V = 4.0 / 3.0 * np.pi * _CUTOFF ** 3
    rdf_out = count / (vol_bins / V)
    return (count, bins, rdf_out)


# speedup vs baseline: 2.0065x; 2.0065x over previous
import functools

import numpy as np
import jax
import jax.numpy as jnp
from jax import lax
from jax.experimental import pallas as pl
from jax.experimental.pallas import tpu as pltpu
from jax.experimental.pallas import tpu_sc as plsc

_NBINS = 100
_CUTOFF = 0.35
_NA = 500
_NAP = 512
_W = _CUTOFF / (_NBINS - 1)
_INVW = (_NBINS - 1) / _CUTOFF
_J = 6
_NH = 128
_R2T = (_CUTOFF + _J * _W) ** 2
_NW = 32
_BUF = 8448
_CW = 2 * _NAP + 16

_mesh = plsc.VectorSubcoreMesh(core_axis_name="c", subcore_axis_name="s")


@functools.partial(
    pl.kernel,
    out_type=jax.ShapeDtypeStruct((_NW * _NH,), jnp.float32),
    mesh=_mesh,
    compiler_params=pltpu.CompilerParams(needs_layout_passes=False),
    scratch_types=[
        pltpu.VMEM((3 * _CW,), jnp.float32),
        pltpu.VMEM((_BUF,), jnp.float32),
        pltpu.VMEM((16 * _NH,), jnp.float32),
        pltpu.VMEM((_NH,), jnp.float32),
    ],
)
def _sc_hist(coords_hbm, out_hbm, cvm, buf, hist, outv):
    wid = lax.axis_index("s") * 2 + lax.axis_index("c")
    pltpu.sync_copy(coords_hbm, cvm)
    iota = lax.iota(jnp.int32, 16)
    zero16 = jnp.zeros((16,), jnp.float32)
    for c in range(16 * _NH // 16):
        hist[pl.ds(c * 16, 16)] = zero16

    def one_batch(b, cursor):
        base_col = b * _NAP
        nrows = (_NA - 1 - wid) // _NW + 1

        def row_body(ri, cur):
            i = wid + _NW * ri
            ci = base_col + i
            civ = jnp.full((16,), ci, jnp.int32)
            xi = plsc.load_gather(cvm, [civ])
            yi = plsc.load_gather(cvm, [civ + _CW])
            zi = plsc.load_gather(cvm, [civ + 2 * _CW])
            nj = (i + 15) // 16

            def jv_body(jv, cur2):
                off = base_col + jv * 16
                jidx = jv * 16 + iota
                dx = xi - cvm[pl.ds(off, 16)]
                dy = yi - cvm[pl.ds(off + _CW, 16)]
                dz = zi - cvm[pl.ds(off + 2 * _CW, 16)]
                dx = (dx + jnp.where(dx >= 0.5, -1.0, 0.0)
                         + jnp.where(dx < -0.5, 1.0, 0.0))
                dy = (dy + jnp.where(dy >= 0.5, -1.0, 0.0)
                         + jnp.where(dy < -0.5, 1.0, 0.0))
                dz = (dz + jnp.where(dz >= 0.5, -1.0, 0.0)
                         + jnp.where(dz < -0.5, 1.0, 0.0))
                dsq = dx * dx + dy * dy + dz * dz
                m = (dsq < _R2T) & (dsq != 0.0) & (jidx < i)
                pos = plsc.cumsum(m.astype(jnp.int32))
                plsc.store_scatter(buf, [cur2 + pos - 1], dsq, mask=m)
                return cur2 + pos[15]

            return lax.fori_loop(0, nj, jv_body, cur)

        return lax.fori_loop(0, nrows, row_body, cursor)

    n = one_batch(0, jnp.int32(0))
    n = one_batch(1, n)

    nv = (n + 15) // 16

    def pv(kv, carry):
        off = kv * 16
        dsq = buf[pl.ds(off, 16)]
        valid = (off + iota) < n
        bits = plsc.bitcast(dsq, jnp.int32)
        y = plsc.bitcast(
            jnp.int32(0x5F3759DF) - lax.shift_right_logical(bits, 1),
            jnp.float32)
        for _ in range(3):
            y = y * (1.5 - 0.5 * dsq * y * y)
        t = dsq * y * _INVW
        i0 = (t + 0.5).astype(jnp.int32)
        f = t - i0.astype(jnp.float32)
        for j in range(-_J, _J + 1):
            a = f - float(j)
            wv = jnp.exp(-0.5 * a * a)
            col = i0 + (j + _J)
            col = jnp.minimum(jnp.maximum(col, 0), _NH - 1)
            plsc.addupdate_scatter(hist, [iota * _NH + col], wv, mask=valid)
        return carry

    lax.fori_loop(0, nv, pv, jnp.int32(0))

    for c in range(8):
        acc = hist[pl.ds(c * 16, 16)]
        for r in range(1, 16):
            acc = acc + hist[pl.ds(r * _NH + c * 16, 16)]
        outv[pl.ds(c * 16, 16)] = acc
    pltpu.sync_copy(outv, out_hbm.at[pl.ds(wid * _NH, _NH)])


def kernel(xyz):
    b = xyz.shape[0]
    xyzp = jnp.pad(xyz, ((0, 0), (0, _NAP - _NA), (0, 0)))
    coords = jnp.transpose(xyzp, (2, 0, 1)).reshape(3, b * _NAP)
    coords = jnp.pad(coords, ((0, 0), (0, 16))).reshape(-1)
    part = _sc_hist(coords).reshape(_NW, _NH)
    count = part.sum(axis=0)[_J:_J + _NBINS]
    bins = jnp.linspace(0.0, _CUTOFF, _NBINS + 1)
    vol_bins = 4.0 * np.pi / 3.0 * (bins[1:] ** 3 - bins[:-1] ** 3)
    norm = count.sum()
    count = count / norm
    V = 4.0 / 3.0 * np.pi * _CUTOFF ** 3
    rdf_out = count / (vol_bins / V)
    return (count, bins, rdf_out)
